# Initial kernel scaffold; baseline (speedup 1.0000x reference)
import jax
import jax.numpy as jnp
from jax.experimental import pallas as pl


def kernel(x, edge_index, batch, Wq1, Wk1, Wv1, Ws1, bq1, bk1, bv1, bs1, Wq2, Wk2, Wv2, Ws2, bq2, bk2, bv2, bs2, Wq3, Wk3, Wv3, Ws3, bq3, bk3, bv3, bs3):
    raise NotImplementedError("write your pallas kernel here")


# speedup vs baseline: 2.1972x; 2.1972x over previous
import functools

import jax
import jax.numpy as jnp
from jax import lax
from jax.experimental import pallas as pl
from jax.experimental.pallas import tpu as pltpu
from jax.experimental.pallas import tpu_sc as plsc

N = 10000
E = 320000
D = 128
G = 64

NC = 2
NS = 16
NW = NC * NS
EPW = E // NW
C = 80
NCHUNK = EPW // C
NP_ = 10240
RPS = NP_ // NS

_INV_SQRT_D = 1.0 / float(D) ** 0.5
RB = 1000


def _edge_body(q_hbm, kv_hbm, src_hbm, dst_hbm, aggp, denp,
               sidx, didx, qb, kvb, wdb, agg_sh, den_sh,
               sem1, sem2):
    c = lax.axis_index("c")
    s = lax.axis_index("s")
    wid = s * NC + c

    def _zrow(i, carry):
        for j in range(8):
            qb[i, pl.ds(j * 16, 16)] = jnp.zeros((16,), jnp.float32)
        wdb[i, :] = jnp.zeros((16,), jnp.float32)
        return carry

    lax.fori_loop(0, C, _zrow, 0)

    for rep in range(RPS // C):
        pltpu.sync_copy(qb, agg_sh.at[pl.ds(s * RPS + rep * C, C), :])
        pltpu.sync_copy(wdb, den_sh.at[pl.ds(s * RPS + rep * C, C), :])
    plsc.subcore_barrier()

    def _chunk(i, carry):
        base = pl.multiple_of(wid * EPW + i * C, 8)
        pltpu.sync_copy(src_hbm.at[pl.ds(base, C)], sidx)
        pltpu.sync_copy(dst_hbm.at[pl.ds(base, C)], didx)
        cp1 = pltpu.async_copy(kv_hbm.at[sidx], kvb, sem1)
        cp2 = pltpu.async_copy(q_hbm.at[didx], qb, sem2)
        cp1.wait()
        cp2.wait()

        iota16 = lax.iota(jnp.int32, 16)
        rows = [jnp.full((16,), g * 16, jnp.int32) + iota16
                for g in range(C // 16)]

        def _alpha_step(i, accs):
            out = accs
            for u in range(4):
                dcol = jnp.full((16,), i * 4 + u, jnp.int32)
                out = tuple(
                    acc + plsc.load_gather(qb, [r, dcol])
                    * plsc.load_gather(kvb, [r, dcol])
                    for acc, r in zip(out, rows))
            return out

        accs = lax.fori_loop(
            0, D // 4, _alpha_step,
            tuple(jnp.zeros((16,), jnp.float32) for _ in range(C // 16)))
        ws = [jnp.exp(a * _INV_SQRT_D) for a in accs]
        zcol = jnp.zeros((16,), jnp.int32)
        for w, r in zip(ws, rows):
            plsc.store_scatter(wdb, [r, zcol], w)

        def _wv_step(i, carry):
            for u in range(4):
                dcol = jnp.full((16,), i * 4 + u, jnp.int32)
                for w, r in zip(ws, rows):
                    v = plsc.load_gather(kvb, [r, dcol + D])
                    plsc.store_scatter(qb, [r, dcol], v * w)
            return carry

        lax.fori_loop(0, D // 4, _wv_step, 0)

        pltpu.sync_copy(qb, agg_sh.at[didx], add=True)
        pltpu.sync_copy(wdb, den_sh.at[didx], add=True)
        return carry

    lax.fori_loop(0, NCHUNK, _chunk, 0)
    plsc.subcore_barrier()

    pltpu.sync_copy(agg_sh.at[pl.ds(s * RPS, RPS), :],
                    aggp.at[c, pl.ds(s * RPS, RPS), :])
    pltpu.sync_copy(den_sh.at[pl.ds(s * RPS, RPS), :],
                    denp.at[c, pl.ds(s * RPS, RPS), :])


_edge_pass = functools.partial(
    pl.kernel,
    out_type=(jax.ShapeDtypeStruct((NC, NP_, D), jnp.float32),
              jax.ShapeDtypeStruct((NC, NP_, 16), jnp.float32)),
    mesh=plsc.VectorSubcoreMesh(core_axis_name="c", subcore_axis_name="s"),
    scratch_types=[
        pltpu.VMEM((C,), jnp.int32),
        pltpu.VMEM((C,), jnp.int32),
        pltpu.VMEM((C, D), jnp.float32),
        pltpu.VMEM((C, 2 * D), jnp.float32),
        pltpu.VMEM((C, 16), jnp.float32),
        pltpu.VMEM_SHARED((NP_, D), jnp.float32),
        pltpu.VMEM_SHARED((NP_, 16), jnp.float32),
        pltpu.SemaphoreType.DMA,
        pltpu.SemaphoreType.DMA,
    ],
    compiler_params=pltpu.CompilerParams(needs_layout_passes=False,
                                         use_tc_tiling_on_sc=False),
)(_edge_body)


def _mm1_body(x_ref, w_ref, b_ref, q_ref, kv_ref, s_ref):
    acc = jnp.dot(x_ref[...], w_ref[...],
                  preferred_element_type=jnp.float32) + b_ref[...]
    q_ref[...] = acc[:, :D]
    kv_ref[...] = acc[:, D:3 * D]
    s_ref[...] = acc[:, 3 * D:]


def _mm1(x, w, b):
    return pl.pallas_call(
        _mm1_body,
        grid=(N // RB,),
        in_specs=[pl.BlockSpec((RB, D), lambda i: (i, 0)),
                  pl.BlockSpec((D, 4 * D), lambda i: (0, 0)),
                  pl.BlockSpec((1, 4 * D), lambda i: (0, 0))],
        out_specs=[pl.BlockSpec((RB, D), lambda i: (i, 0)),
                   pl.BlockSpec((RB, 2 * D), lambda i: (i, 0)),
                   pl.BlockSpec((RB, D), lambda i: (i, 0))],
        out_shape=[jax.ShapeDtypeStruct((N, D), jnp.float32),
                   jax.ShapeDtypeStruct((N, 2 * D), jnp.float32),
                   jax.ShapeDtypeStruct((N, D), jnp.float32)],
    )(x, w, b)


def _norm_h(aggp_ref, denp_ref, s_ref):
    agg = aggp_ref[0] + aggp_ref[1]
    den = denp_ref[0][:, :1] + denp_ref[1][:, :1]
    return jnp.maximum(agg / jnp.maximum(den, 1e-30) + s_ref[...], 0.0)


def _mmf_body(aggp_ref, denp_ref, s_ref, w_ref, b_ref, q_ref, kv_ref, so_ref):
    h = _norm_h(aggp_ref, denp_ref, s_ref)
    acc = jnp.dot(h, w_ref[...],
                  preferred_element_type=jnp.float32) + b_ref[...]
    q_ref[...] = acc[:, :D]
    kv_ref[...] = acc[:, D:3 * D]
    so_ref[...] = acc[:, 3 * D:]


def _mmf(aggp, denp, s_prev, w, b):
    return pl.pallas_call(
        _mmf_body,
        grid=(N // RB,),
        in_specs=[pl.BlockSpec((NC, RB, D), lambda i: (0, i, 0)),
                  pl.BlockSpec((NC, RB, 16), lambda i: (0, i, 0)),
                  pl.BlockSpec((RB, D), lambda i: (i, 0)),
                  pl.BlockSpec((D, 4 * D), lambda i: (0, 0)),
                  pl.BlockSpec((1, 4 * D), lambda i: (0, 0))],
        out_specs=[pl.BlockSpec((RB, D), lambda i: (i, 0)),
                   pl.BlockSpec((RB, 2 * D), lambda i: (i, 0)),
                   pl.BlockSpec((RB, D), lambda i: (i, 0))],
        out_shape=[jax.ShapeDtypeStruct((N, D), jnp.float32),
                   jax.ShapeDtypeStruct((N, 2 * D), jnp.float32),
                   jax.ShapeDtypeStruct((N, D), jnp.float32)],
    )(aggp, denp, s_prev, w, b)


def _pool_body(aggp_ref, denp_ref, s_ref, b_ref, out_ref, sums, counts):
    i = pl.program_id(0)
    h = _norm_h(aggp_ref, denp_ref, s_ref)
    bids = b_ref[0, 0, :]
    oh = (lax.broadcasted_iota(jnp.int32, (G, RB), 0)
          == bids[None, :]).astype(jnp.float32)
    ps = jnp.dot(oh, h, preferred_element_type=jnp.float32)
    pc = jnp.dot(oh, jnp.ones((RB, D), jnp.float32),
                 preferred_element_type=jnp.float32)

    @pl.when(i == 0)
    def _():
        sums[...] = jnp.zeros_like(sums)
        counts[...] = jnp.zeros_like(counts)

    sums[...] += ps
    counts[...] += pc

    @pl.when(i == pl.num_programs(0) - 1)
    def _():
        out_ref[...] = sums[...] / jnp.maximum(counts[...], 1.0)


def _pool(aggp, denp, s_prev, batch3):
    return pl.pallas_call(
        _pool_body,
        grid=(N // RB,),
        in_specs=[pl.BlockSpec((NC, RB, D), lambda i: (0, i, 0)),
                  pl.BlockSpec((NC, RB, 16), lambda i: (0, i, 0)),
                  pl.BlockSpec((RB, D), lambda i: (i, 0)),
                  pl.BlockSpec((1, 1, RB), lambda i: (i, 0, 0))],
        out_specs=pl.BlockSpec((G, D), lambda i: (0, 0)),
        out_shape=jax.ShapeDtypeStruct((G, D), jnp.float32),
        scratch_shapes=[pltpu.VMEM((G, D), jnp.float32),
                        pltpu.VMEM((G, D), jnp.float32)],
    )(aggp, denp, s_prev, batch3)


def _pack(Wq, Wk, Wv, Ws, bq, bk, bv, bs):
    w = jnp.concatenate([Wq, Wk, Wv, Ws], axis=1)
    b = jnp.concatenate([bq, bk, bv, bs]).reshape(1, 4 * D)
    return w, b


def kernel(x, edge_index, batch,
           Wq1, Wk1, Wv1, Ws1, bq1, bk1, bv1, bs1,
           Wq2, Wk2, Wv2, Ws2, bq2, bk2, bv2, bs2,
           Wq3, Wk3, Wv3, Ws3, bq3, bk3, bv3, bs3):
    src = edge_index[0]
    dst = edge_index[1]
    batch3 = batch.reshape(N // RB, 1, RB).astype(jnp.int32)
    w1, b1 = _pack(Wq1, Wk1, Wv1, Ws1, bq1, bk1, bv1, bs1)
    w2, b2 = _pack(Wq2, Wk2, Wv2, Ws2, bq2, bk2, bv2, bs2)
    w3, b3 = _pack(Wq3, Wk3, Wv3, Ws3, bq3, bk3, bv3, bs3)

    q, kv, s = _mm1(x, w1, b1)
    aggp, denp = _edge_pass(q, kv, src, dst)
    q, kv, s = _mmf(aggp, denp, s, w2, b2)
    aggp, denp = _edge_pass(q, kv, src, dst)
    q, kv, s = _mmf(aggp, denp, s, w3, b3)
    aggp, denp = _edge_pass(q, kv, src, dst)
    return _pool(aggp, denp, s, batch3)


# speedup vs baseline: 3.9564x; 1.8007x over previous
import functools

import jax
import jax.numpy as jnp
from jax import lax
from jax.experimental import pallas as pl
from jax.experimental.pallas import tpu as pltpu
from jax.experimental.pallas import tpu_sc as plsc

N = 10000
E = 320000
D = 128
G = 64
DW = D // 2

NC = 2
NS = 16
NW = NC * NS
EPW = E // NW
C = 80
NG = C // 16
NCHUNK = EPW // C
NP_ = 10240
RPS = NP_ // NS
DRPS = NP_ // 16 // NS

_INV_SQRT_D = 1.0 / float(D) ** 0.5
RB = 1000


def _edge_body(qp_hbm, kvp_hbm, src_hbm, dst_hbm, aggp, denp,
               sidx0, sidx1, didx0, didx1, dscb0, dscb1, dshb0, dshb1,
               qb0, qb1, kvb0, kvb1, wvb, wdb, agg_sh, den_sh,
               semi0, semi1, semg0, semg1, sems0, sems1):
    c = lax.axis_index("c")
    s = lax.axis_index("s")
    wid = s * NC + c
    ebase = wid * EPW

    sidx = [sidx0, sidx1]
    didx = [didx0, didx1]
    dscb = [dscb0, dscb1]
    dshb = [dshb0, dshb1]
    qb = [qb0, qb1]
    kvb = [kvb0, kvb1]
    semi = [semi0, semi1]
    semg = [semg0, semg1]
    sems = [sems0, sems1]

    def _zrow(i, carry):
        for j in range(8):
            wvb[i, pl.ds(j * 16, 16)] = jnp.zeros((16,), jnp.float32)
        wdb[i, :] = jnp.zeros((16,), jnp.float32)
        return carry

    lax.fori_loop(0, C, _zrow, 0)
    for rep in range(RPS // C):
        pltpu.sync_copy(wvb, agg_sh.at[pl.ds(s * RPS + rep * C, C), :])
    pltpu.sync_copy(wdb.at[pl.ds(0, DRPS), :],
                    den_sh.at[pl.ds(s * DRPS, DRPS), :])
    plsc.subcore_barrier()

    iota16 = lax.iota(jnp.int32, 16)
    rows = [jnp.full((16,), g * 16, jnp.int32) + iota16 for g in range(NG)]
    MHI = jnp.int32(-65536)

    def _issue_idx(pi, i1):
        b = pl.multiple_of(ebase + i1 * C, 8)
        pltpu.async_copy(src_hbm.at[pl.ds(b, C)], sidx[pi], semi[pi])
        pltpu.async_copy(dst_hbm.at[pl.ds(b, C)], didx[pi], semi[pi])

    def _wait_idx(pi):
        pltpu.make_async_copy(src_hbm.at[pl.ds(0, C)], sidx[pi], semi[pi]).wait()
        pltpu.make_async_copy(dst_hbm.at[pl.ds(0, C)], didx[pi], semi[pi]).wait()

    def _issue_gather(pi):
        pltpu.async_copy(qp_hbm.at[didx[pi]], qb[pi], semg[pi])
        pltpu.async_copy(kvp_hbm.at[sidx[pi]], kvb[pi], semg[pi])

    def _wait_gather(pi):
        pltpu.make_async_copy(qp_hbm.at[didx[pi]], qb[pi], semg[pi]).wait()
        pltpu.make_async_copy(kvp_hbm.at[sidx[pi]], kvb[pi], semg[pi]).wait()

    def _issue_scatter(pi):
        pltpu.async_copy(wvb, agg_sh.at[dscb[pi]], sems[pi], add=True)
        pltpu.async_copy(wdb, den_sh.at[dshb[pi]], sems[pi], add=True)

    def _wait_scatter(pi):
        pltpu.make_async_copy(wvb, agg_sh.at[dscb[pi]], sems[pi]).wait()
        pltpu.make_async_copy(wdb, den_sh.at[dshb[pi]], sems[pi]).wait()

    def _alpha(pi):
        qc, kc = qb[pi], kvb[pi]

        def _alpha_step(t, accs):
            a = list(accs)
            for u in range(4):
                dcol = jnp.full((16,), t * 4 + u, jnp.int32)
                for g in range(NG):
                    qw = plsc.bitcast(
                        plsc.load_gather(qc, [rows[g], dcol]), jnp.int32)
                    kw = plsc.bitcast(
                        plsc.load_gather(kc, [rows[g], dcol]), jnp.int32)
                    qlo = plsc.bitcast(qw << 16, jnp.float32)
                    klo = plsc.bitcast(kw << 16, jnp.float32)
                    qhi = plsc.bitcast(qw & MHI, jnp.float32)
                    khi = plsc.bitcast(kw & MHI, jnp.float32)
                    a[2 * g] = a[2 * g] + qlo * klo
                    a[2 * g + 1] = a[2 * g + 1] + qhi * khi
            return tuple(a)

        z = jnp.zeros((16,), jnp.float32)
        accs = lax.fori_loop(0, DW // 4, _alpha_step, (z,) * (2 * NG))
        return [jnp.exp((accs[2 * g] + accs[2 * g + 1]) * _INV_SQRT_D)
                for g in range(NG)]

    def _fill(pi, ws):
        for i in range(C):
            wdb[i, :] = jnp.zeros((16,), jnp.float32)
        for g in range(NG):
            dv = didx[pi][pl.ds(g * 16, 16)]
            dscb[pi][pl.ds(g * 16, 16)] = dv
            dshb[pi][pl.ds(g * 16, 16)] = dv >> 4
            plsc.store_scatter(wdb, [rows[g], dv & 15], ws[g])
        kc = kvb[pi]

        def _wv_step(t, carry):
            for u in range(4):
                d0 = t * 4 + u
                dcol = jnp.full((16,), d0, jnp.int32)
                dcol2 = jnp.full((16,), d0 + DW, jnp.int32)
                for g in range(NG):
                    vw = plsc.bitcast(
                        plsc.load_gather(kc, [rows[g], dcol2]), jnp.int32)
                    vlo = plsc.bitcast(vw << 16, jnp.float32)
                    vhi = plsc.bitcast(vw & MHI, jnp.float32)
                    plsc.store_scatter(wvb, [rows[g], dcol], vlo * ws[g])
                    plsc.store_scatter(wvb, [rows[g], dcol2], vhi * ws[g])
            return carry

        lax.fori_loop(0, DW // 4, _wv_step, 0)

    b0 = pl.multiple_of(ebase, 8)
    pltpu.sync_copy(src_hbm.at[pl.ds(b0, C)], sidx0)
    pltpu.sync_copy(dst_hbm.at[pl.ds(b0, C)], didx0)
    _issue_gather(0)

    def _chunk(i, carry):
        nxt_ok = i + 1 < NCHUNK

        def _one(pi):
            pn = 1 - pi

            @pl.when(nxt_ok)
            def _():
                _issue_idx(pn, i + 1)

            _wait_gather(pi)
            ws = _alpha(pi)

            @pl.when(nxt_ok)
            def _():
                _wait_idx(pn)
                _issue_gather(pn)

            @pl.when(i > 0)
            def _():
                _wait_scatter(pn)

            _fill(pi, ws)
            _issue_scatter(pi)

        @pl.when(i % 2 == 0)
        def _():
            _one(0)

        @pl.when(i % 2 == 1)
        def _():
            _one(1)

        return carry

    lax.fori_loop(0, NCHUNK, _chunk, 0)
    _wait_scatter((NCHUNK - 1) % 2)
    plsc.subcore_barrier()

    pltpu.sync_copy(agg_sh.at[pl.ds(s * RPS, RPS), :],
                    aggp.at[c, pl.ds(s * RPS, RPS), :])
    pltpu.sync_copy(den_sh.at[pl.ds(s * DRPS, DRPS), :],
                    denp.at[c, pl.ds(s * DRPS, DRPS), :])


_edge_pass = functools.partial(
    pl.kernel,
    out_type=(jax.ShapeDtypeStruct((NC, NP_, D), jnp.float32),
              jax.ShapeDtypeStruct((NC, NP_ // 16, 16), jnp.float32)),
    mesh=plsc.VectorSubcoreMesh(core_axis_name="c", subcore_axis_name="s"),
    scratch_types=[
        pltpu.VMEM((C,), jnp.int32),
        pltpu.VMEM((C,), jnp.int32),
        pltpu.VMEM((C,), jnp.int32),
        pltpu.VMEM((C,), jnp.int32),
        pltpu.VMEM((C,), jnp.int32),
        pltpu.VMEM((C,), jnp.int32),
        pltpu.VMEM((C,), jnp.int32),
        pltpu.VMEM((C,), jnp.int32),
        pltpu.VMEM((C, DW), jnp.float32),
        pltpu.VMEM((C, DW), jnp.float32),
        pltpu.VMEM((C, D), jnp.float32),
        pltpu.VMEM((C, D), jnp.float32),
        pltpu.VMEM((C, D), jnp.float32),
        pltpu.VMEM((C, 16), jnp.float32),
        pltpu.VMEM_SHARED((NP_, D), jnp.float32),
        pltpu.VMEM_SHARED((NP_ // 16, 16), jnp.float32),
        pltpu.SemaphoreType.DMA,
        pltpu.SemaphoreType.DMA,
        pltpu.SemaphoreType.DMA,
        pltpu.SemaphoreType.DMA,
        pltpu.SemaphoreType.DMA,
        pltpu.SemaphoreType.DMA,
    ],
    compiler_params=pltpu.CompilerParams(needs_layout_passes=False,
                                         use_tc_tiling_on_sc=False),
)(_edge_body)


def _pack_cols(lo, hi):
    lo16 = jax.lax.bitcast_convert_type(
        lo.astype(jnp.bfloat16), jnp.uint16).astype(jnp.uint32)
    hi16 = jax.lax.bitcast_convert_type(
        hi.astype(jnp.bfloat16), jnp.uint16).astype(jnp.uint32)
    return jax.lax.bitcast_convert_type(lo16 | (hi16 << 16), jnp.float32)


def _emit_packed(acc, qp_ref, kvp_ref, s_ref):
    q = acc[:, :D]
    k = acc[:, D:2 * D]
    v = acc[:, 2 * D:3 * D]
    qp_ref[...] = _pack_cols(q[:, :DW], q[:, DW:])
    kvp_ref[...] = jnp.concatenate(
        [_pack_cols(k[:, :DW], k[:, DW:]),
         _pack_cols(v[:, :DW], v[:, DW:])], axis=1)
    s_ref[...] = acc[:, 3 * D:]


def _mm1_body(x_ref, w_ref, b_ref, qp_ref, kvp_ref, s_ref):
    acc = jnp.dot(x_ref[...], w_ref[...],
                  preferred_element_type=jnp.float32) + b_ref[...]
    _emit_packed(acc, qp_ref, kvp_ref, s_ref)


_MM_OUT_SPECS = [pl.BlockSpec((RB, DW), lambda i: (i, 0)),
                 pl.BlockSpec((RB, D), lambda i: (i, 0)),
                 pl.BlockSpec((RB, D), lambda i: (i, 0))]
_MM_OUT_SHAPE = [jax.ShapeDtypeStruct((N, DW), jnp.float32),
                 jax.ShapeDtypeStruct((N, D), jnp.float32),
                 jax.ShapeDtypeStruct((N, D), jnp.float32)]


def _mm1(x, w, b):
    return pl.pallas_call(
        _mm1_body,
        grid=(N // RB,),
        in_specs=[pl.BlockSpec((RB, D), lambda i: (i, 0)),
                  pl.BlockSpec((D, 4 * D), lambda i: (0, 0)),
                  pl.BlockSpec((1, 4 * D), lambda i: (0, 0))],
        out_specs=_MM_OUT_SPECS,
        out_shape=_MM_OUT_SHAPE,
    )(x, w, b)


def _norm_h(aggp_ref, denp_ref, s_ref):
    agg = aggp_ref[0] + aggp_ref[1]
    den = denp_ref[0] + denp_ref[1]
    return jnp.maximum(agg / jnp.maximum(den, 1e-30) + s_ref[...], 0.0)


def _mmf_body(aggp_ref, denp_ref, s_ref, w_ref, b_ref, qp_ref, kvp_ref,
              so_ref):
    h = _norm_h(aggp_ref, denp_ref, s_ref)
    acc = jnp.dot(h, w_ref[...],
                  preferred_element_type=jnp.float32) + b_ref[...]
    _emit_packed(acc, qp_ref, kvp_ref, so_ref)


def _mmf(aggp, denp3, s_prev, w, b):
    return pl.pallas_call(
        _mmf_body,
        grid=(N // RB,),
        in_specs=[pl.BlockSpec((NC, RB, D), lambda i: (0, i, 0)),
                  pl.BlockSpec((NC, RB, 1), lambda i: (0, i, 0)),
                  pl.BlockSpec((RB, D), lambda i: (i, 0)),
                  pl.BlockSpec((D, 4 * D), lambda i: (0, 0)),
                  pl.BlockSpec((1, 4 * D), lambda i: (0, 0))],
        out_specs=_MM_OUT_SPECS,
        out_shape=_MM_OUT_SHAPE,
    )(aggp, denp3, s_prev, w, b)


def _pool_body(aggp_ref, denp_ref, s_ref, b_ref, out_ref, sums, counts):
    i = pl.program_id(0)
    h = _norm_h(aggp_ref, denp_ref, s_ref)
    bids = b_ref[0, 0, :]
    oh = (lax.broadcasted_iota(jnp.int32, (G, RB), 0)
          == bids[None, :]).astype(jnp.float32)
    ps = jnp.dot(oh, h, preferred_element_type=jnp.float32)
    pc = jnp.dot(oh, jnp.ones((RB, D), jnp.float32),
                 preferred_element_type=jnp.float32)

    @pl.when(i == 0)
    def _():
        sums[...] = jnp.zeros_like(sums)
        counts[...] = jnp.zeros_like(counts)

    sums[...] += ps
    counts[...] += pc

    @pl.when(i == pl.num_programs(0) - 1)
    def _():
        out_ref[...] = sums[...] / jnp.maximum(counts[...], 1.0)


def _pool(aggp, denp3, s_prev, batch3):
    return pl.pallas_call(
        _pool_body,
        grid=(N // RB,),
        in_specs=[pl.BlockSpec((NC, RB, D), lambda i: (0, i, 0)),
                  pl.BlockSpec((NC, RB, 1), lambda i: (0, i, 0)),
                  pl.BlockSpec((RB, D), lambda i: (i, 0)),
                  pl.BlockSpec((1, 1, RB), lambda i: (i, 0, 0))],
        out_specs=pl.BlockSpec((G, D), lambda i: (0, 0)),
        out_shape=jax.ShapeDtypeStruct((G, D), jnp.float32),
        scratch_shapes=[pltpu.VMEM((G, D), jnp.float32),
                        pltpu.VMEM((G, D), jnp.float32)],
    )(aggp, denp3, s_prev, batch3)


def _pack_w(Wq, Wk, Wv, Ws, bq, bk, bv, bs):
    w = jnp.concatenate([Wq, Wk, Wv, Ws], axis=1)
    b = jnp.concatenate([bq, bk, bv, bs]).reshape(1, 4 * D)
    return w, b


def kernel(x, edge_index, batch,
           Wq1, Wk1, Wv1, Ws1, bq1, bk1, bv1, bs1,
           Wq2, Wk2, Wv2, Ws2, bq2, bk2, bv2, bs2,
           Wq3, Wk3, Wv3, Ws3, bq3, bk3, bv3, bs3):
    src = edge_index[0]
    dst = edge_index[1]
    batch3 = batch.reshape(N // RB, 1, RB).astype(jnp.int32)
    w1, b1 = _pack_w(Wq1, Wk1, Wv1, Ws1, bq1, bk1, bv1, bs1)
    w2, b2 = _pack_w(Wq2, Wk2, Wv2, Ws2, bq2, bk2, bv2, bs2)
    w3, b3 = _pack_w(Wq3, Wk3, Wv3, Ws3, bq3, bk3, bv3, bs3)

    qp, kvp, s = _mm1(x, w1, b1)
    aggp, denp = _edge_pass(qp, kvp, src, dst)
    q2 = _mmf(aggp, denp.reshape(NC, NP_, 1), s, w2, b2)
    qp, kvp, s = q2
    aggp, denp = _edge_pass(qp, kvp, src, dst)
    qp, kvp, s = _mmf(aggp, denp.reshape(NC, NP_, 1), s, w3, b3)
    aggp, denp = _edge_pass(qp, kvp, src, dst)
    return _pool(aggp, denp.reshape(NC, NP_, 1), s, batch3)
